# baseline (speedup 1.0000x reference)
import functools
import math

import jax
import jax.numpy as jnp
from jax import lax
from jax.experimental import pallas as pl
from jax.experimental.pallas import tpu as pltpu
from jax.experimental.pallas import tpu_sc as plsc

D_MODEL = 64
SCALE = math.sqrt(D_MODEL)
NUM_WORKERS = 32


def _sc_embed(x, table):
    batch, seq = x.shape
    b_per_w = batch // NUM_WORKERS
    mesh = plsc.VectorSubcoreMesh(core_axis_name="c", subcore_axis_name="s")

    @functools.partial(
        pl.kernel,
        out_type=jax.ShapeDtypeStruct((batch, seq, D_MODEL), jnp.float32),
        mesh=mesh,
        scratch_types=[
            pltpu.VMEM((b_per_w, seq), jnp.int32),
            pltpu.VMEM((seq, D_MODEL), jnp.float32),
            pltpu.VMEM((seq, D_MODEL), jnp.float32),
            pltpu.SemaphoreType.DMA,
            pltpu.SemaphoreType.DMA,
            pltpu.SemaphoreType.DMA,
            pltpu.SemaphoreType.DMA,
        ],
        compiler_params=pltpu.CompilerParams(use_tc_tiling_on_sc=False),
    )
    def k(x_hbm, table_hbm, out_hbm, idx_slab, rows0, rows1,
          gsem0, gsem1, ssem0, ssem1):
        rows = (rows0, rows1)
        gsem = (gsem0, gsem1)
        ssem = (ssem0, ssem1)
        wid = lax.axis_index("s") * 2 + lax.axis_index("c")
        base = wid * b_per_w

        pltpu.sync_copy(x_hbm.at[pl.ds(base, b_per_w)], idx_slab)

        def start_gather(b, buf):
            pltpu.async_copy(table_hbm.at[idx_slab.at[b]], rows[buf],
                             gsem[buf])

        def scale_buf(buf):
            def scale_row(i, carry2):
                for j in range(D_MODEL // 16):
                    s = pl.ds(j * 16, 16)
                    rows[buf][i, s] = rows[buf][i, s] * SCALE
                return carry2
            lax.fori_loop(0, seq, scale_row, 0, unroll=4)

        def start_store(b, buf):
            pltpu.async_copy(rows[buf], out_hbm.at[base + b], ssem[buf])

        def wait_gather(buf):
            pltpu.make_async_copy(table_hbm.at[idx_slab.at[0]], rows[buf],
                                  gsem[buf]).wait()

        def wait_store(buf):
            pltpu.make_async_copy(rows[buf], out_hbm.at[0], ssem[buf]).wait()

        start_gather(0, 0)

        def outer(g, carry):
            for buf in (0, 1):
                b = 2 * g + buf
                other = 1 - buf
                wait_gather(buf)
                if buf == 1:
                    wait_store(other)
                else:
                    @pl.when(g > 0)
                    def _():
                        wait_store(other)
                if buf == 0:
                    start_gather(b + 1, other)
                else:
                    @pl.when(2 * g + 2 < b_per_w)
                    def _():
                        start_gather(b + 1, other)
                scale_buf(buf)
                start_store(b, buf)
            return carry

        lax.fori_loop(0, b_per_w // 2, outer, 0)
        wait_store(1)

    return k(x, table)


def kernel(x, table):
    return _sc_embed(x, table)
